# baseline (speedup 1.0000x reference)
import functools

import jax
import jax.numpy as jnp
from jax import lax
from jax.experimental import pallas as pl
from jax.experimental.pallas import tpu as pltpu
from jax.experimental.pallas import tpu_sc as plsc

N_TOK = 4096
D_IN = 768
D_HID = 512
D_OUT = 768
N_EXP = 8
BLK = 512
LANES = 128
N_HALVES = 2
HALF = N_TOK // N_HALVES


def _shift_lanes_right(a, k):
    return jnp.concatenate([jnp.zeros((1, k), a.dtype), a[:, :-k]], axis=1)


def _router_rank_kernel(n_blk, x_ref, wg_ref, bg_ref, rank_ref, hist_ref,
                        onehot_scr, carry_scr, bcarry_scr):
    i = pl.program_id(0)
    b = lax.rem(i, n_blk)

    @pl.when(i == 0)
    def _init():
        carry_scr[...] = jnp.zeros((1, LANES), jnp.float32)

    @pl.when(i < n_blk)
    def _phase0():
        logits = jnp.dot(x_ref[...], wg_ref[...],
                         preferred_element_type=jnp.float32) + bg_ref[...]
        m = jnp.max(logits, axis=1, keepdims=True)
        lane = lax.broadcasted_iota(jnp.int32, (BLK, LANES), 1)
        eid = jnp.min(jnp.where(logits == m, lane, LANES), axis=1,
                      keepdims=True)
        onehot = (lane == eid).astype(jnp.float32)
        onehot_scr[pl.ds(b * BLK, BLK), :] = onehot
        bcarry_scr[pl.ds(b, 1), :] = carry_scr[...]
        carry_scr[...] = carry_scr[...] + jnp.sum(onehot, axis=0,
                                                  keepdims=True)

    @pl.when(i >= n_blk)
    def _phase1():
        onehot = onehot_scr[pl.ds(b * BLK, BLK), :]
        hist = carry_scr[...]
        incl = hist
        for k in (1, 2, 4):
            incl = incl + _shift_lanes_right(incl, k)
        start = incl - hist
        row = lax.broadcasted_iota(jnp.int32, (BLK, BLK), 0)
        col = lax.broadcasted_iota(jnp.int32, (BLK, BLK), 1)
        tril = (row >= col).astype(jnp.float32)
        within_incl = jnp.dot(tril, onehot,
                              preferred_element_type=jnp.float32)
        rank_f = jnp.sum(
            (within_incl + bcarry_scr[pl.ds(b, 1), :] + start) * onehot,
            axis=1, keepdims=True) - 1.0
        rank_ref[...] = rank_f.astype(jnp.int32)

    hist_ref[...] = carry_scr[...].astype(jnp.int32)


def _router_rank(x, wg_pad, bg_pad):
    n_tok = x.shape[0]
    n_blk = n_tok // BLK
    return pl.pallas_call(
        functools.partial(_router_rank_kernel, n_blk),
        grid=(2 * n_blk,),
        in_specs=[
            pl.BlockSpec((BLK, D_IN), lambda i, n=n_blk: (lax.rem(i, n), 0)),
            pl.BlockSpec((D_IN, LANES), lambda i: (0, 0)),
            pl.BlockSpec((1, LANES), lambda i: (0, 0)),
        ],
        out_specs=[
            pl.BlockSpec((BLK, 1), lambda i, n=n_blk: (lax.rem(i, n), 0)),
            pl.BlockSpec((1, LANES), lambda i: (0, 0)),
        ],
        out_shape=[
            jax.ShapeDtypeStruct((n_tok, 1), jnp.int32),
            jax.ShapeDtypeStruct((1, LANES), jnp.int32),
        ],
        scratch_shapes=[
            pltpu.VMEM((n_tok, LANES), jnp.float32),
            pltpu.VMEM((1, LANES), jnp.float32),
            pltpu.VMEM((n_blk, LANES), jnp.float32),
        ],
    )(x, wg_pad, bg_pad)


def _group_gemm_kernel(hist_ref, xs_ref, w1_ref, b1_ref, wout_ref,
                       bout_ref, out_ref, acc_scr):
    tile_base = pl.program_id(0) * BLK
    starts = [jnp.int32(0)]
    for e in range(N_EXP):
        starts.append(starts[-1] + hist_ref[e])
    r = lax.broadcasted_iota(jnp.int32, (BLK, 1), 0) + tile_base
    for e in range(N_EXP):
        present = jnp.logical_and(starts[e + 1] > tile_base,
                                  starts[e] < tile_base + BLK)

        @pl.when(present)
        def _run(e=e):
            h = jnp.maximum(
                jnp.dot(xs_ref[...], w1_ref[e],
                        preferred_element_type=jnp.float32)
                + b1_ref[e][None, :], 0.0)
            mask = jnp.logical_and(r >= starts[e], r < starts[e + 1])
            acc_scr[...] = jnp.where(mask, h, acc_scr[...])

    out_ref[...] = jnp.dot(acc_scr[...], wout_ref[...],
                           preferred_element_type=jnp.float32) + bout_ref[...]


def _group_gemm(hist, xs, W1, b1, Wout, bout2d):
    n_tok = xs.shape[0]
    n_blk = n_tok // BLK
    return pl.pallas_call(
        _group_gemm_kernel,
        grid=(n_blk,),
        in_specs=[
            pl.BlockSpec(memory_space=pltpu.SMEM),
            pl.BlockSpec((BLK, D_IN), lambda i: (i, 0)),
            pl.BlockSpec((N_EXP, D_IN, D_HID), lambda i: (0, 0, 0)),
            pl.BlockSpec((N_EXP, D_HID), lambda i: (0, 0)),
            pl.BlockSpec((D_HID, D_OUT), lambda i: (0, 0)),
            pl.BlockSpec((1, D_OUT), lambda i: (0, 0)),
        ],
        out_specs=pl.BlockSpec((BLK, D_OUT), lambda i: (i, 0)),
        out_shape=jax.ShapeDtypeStruct((n_tok, D_OUT), jnp.float32),
        scratch_shapes=[pltpu.VMEM((BLK, D_HID), jnp.float32)],
    )(hist, xs, W1, b1, Wout, bout2d)


_NC = 2
_NS = 16
_NW = _NC * _NS


@functools.lru_cache(maxsize=None)
def _make_sc_permute(n_tok, d_model, invert):
    rpw = n_tok // _NW
    mesh = plsc.VectorSubcoreMesh(core_axis_name="c", subcore_axis_name="s")

    @functools.partial(
        pl.kernel, mesh=mesh,
        out_type=jax.ShapeDtypeStruct((n_tok, d_model), jnp.float32),
        scratch_types=[
            pltpu.VMEM((rpw,), jnp.int32),
            pltpu.VMEM((rpw, d_model), jnp.float32),
            pltpu.SemaphoreType.DMA,
        ],
    )
    def k(src_hbm, idx_hbm, out_hbm, idx_v, rows_v, sem):
        wid = lax.axis_index("s") * _NC + lax.axis_index("c")
        base = wid * rpw
        pltpu.sync_copy(idx_hbm.at[pl.ds(base, rpw)], idx_v)
        if invert:
            pltpu.async_copy(src_hbm.at[idx_v], rows_v, sem).wait()
            pltpu.sync_copy(rows_v, out_hbm.at[pl.ds(base, rpw)])
        else:
            pltpu.sync_copy(src_hbm.at[pl.ds(base, rpw)], rows_v)
            pltpu.async_copy(rows_v, out_hbm.at[idx_v], sem).wait()

    return k


def kernel(x, Wg, bg, W1, b1, Wout, bout):
    wg_pad = jnp.concatenate(
        [Wg, jnp.zeros((D_IN, LANES - N_EXP), Wg.dtype)], axis=1)
    bg_pad = jnp.concatenate(
        [bg, jnp.full((LANES - N_EXP,), -1e30, bg.dtype)]).reshape(1, LANES)
    bout2d = bout.reshape(1, D_OUT)
    scatter = _make_sc_permute(HALF, D_IN, False)
    gather = _make_sc_permute(HALF, D_OUT, True)

    ranks, hists = [], []
    for hh in range(N_HALVES):
        r2d, h2d = _router_rank(x[hh * HALF:(hh + 1) * HALF], wg_pad, bg_pad)
        ranks.append(r2d.reshape(HALF))
        hists.append(h2d.reshape(LANES)[:N_EXP])
    xss = [scatter(x[hh * HALF:(hh + 1) * HALF], ranks[hh])
           for hh in range(N_HALVES)]
    oss = [_group_gemm(hists[hh], xss[hh], W1, b1, Wout, bout2d)
           for hh in range(N_HALVES)]
    outs = [gather(oss[hh], ranks[hh]) for hh in range(N_HALVES)]
    return jnp.concatenate(outs, axis=0)


# speedup vs baseline: 1.4723x; 1.4723x over previous
import functools

import jax
import jax.numpy as jnp
from jax import lax
from jax.experimental import pallas as pl
from jax.experimental.pallas import tpu as pltpu
from jax.experimental.pallas import tpu_sc as plsc

N_TOK = 4096
D_IN = 768
D_HID = 512
D_OUT = 768
N_EXP = 8
BLK = 512
N_BLK = N_TOK // BLK
LANES = 128
RROWS = BLK // LANES


def _shift_lanes_right(a, k):
    return jnp.concatenate([jnp.zeros((1, k), a.dtype), a[:, :-k]], axis=1)


def _router_rank_kernel(x_ref, wg_ref, bg_ref, rank_ref, hist_ref,
                        onehot_scr, carry_scr, bcarry_scr):
    i = pl.program_id(0)
    b = lax.rem(i, N_BLK)

    @pl.when(i == 0)
    def _init():
        carry_scr[...] = jnp.zeros((1, LANES), jnp.float32)

    @pl.when(i < N_BLK)
    def _phase0():
        logits8 = jnp.dot(x_ref[...], wg_ref[...],
                          preferred_element_type=jnp.float32) + bg_ref[...]
        logits = jnp.concatenate(
            [logits8, jnp.full((BLK, LANES - N_EXP), -1e30, jnp.float32)],
            axis=1)
        m = jnp.max(logits, axis=1, keepdims=True)
        lane = lax.broadcasted_iota(jnp.int32, (BLK, LANES), 1)
        eid = jnp.min(jnp.where(logits == m, lane, LANES), axis=1,
                      keepdims=True)
        onehot = (lane == eid).astype(jnp.float32)
        onehot_scr[pl.ds(b * BLK, BLK), :] = onehot
        bcarry_scr[pl.ds(b, 1), :] = carry_scr[...]
        carry_scr[...] = carry_scr[...] + jnp.sum(onehot, axis=0,
                                                  keepdims=True)

    @pl.when(i >= N_BLK)
    def _phase1():
        onehot = onehot_scr[pl.ds(b * BLK, BLK), :]
        hist = carry_scr[...]
        incl = hist
        for k in (1, 2, 4):
            incl = incl + _shift_lanes_right(incl, k)
        start = incl - hist
        row = lax.broadcasted_iota(jnp.int32, (BLK, BLK), 0)
        col = lax.broadcasted_iota(jnp.int32, (BLK, BLK), 1)
        tril = (row >= col).astype(jnp.float32)
        within_incl = jnp.dot(tril, onehot,
                              preferred_element_type=jnp.float32)
        rank_f = jnp.sum(
            (within_incl + bcarry_scr[pl.ds(b, 1), :] + start) * onehot,
            axis=1, keepdims=True) - 1.0
        rank_ref[...] = rank_f.astype(jnp.int32).reshape(1, RROWS, LANES)

    hist_ref[...] = carry_scr[...].astype(jnp.int32)


def _router_rank(x, Wg, bg2d):
    return pl.pallas_call(
        _router_rank_kernel,
        grid=(2 * N_BLK,),
        in_specs=[
            pl.BlockSpec((BLK, D_IN), lambda i: (lax.rem(i, N_BLK), 0)),
            pl.BlockSpec((D_IN, N_EXP), lambda i: (0, 0)),
            pl.BlockSpec((1, N_EXP), lambda i: (0, 0)),
        ],
        out_specs=[
            pl.BlockSpec((1, RROWS, LANES),
                         lambda i: (lax.rem(i, N_BLK), 0, 0)),
            pl.BlockSpec((1, LANES), lambda i: (0, 0)),
        ],
        out_shape=[
            jax.ShapeDtypeStruct((N_BLK, RROWS, LANES), jnp.int32),
            jax.ShapeDtypeStruct((1, LANES), jnp.int32),
        ],
        scratch_shapes=[
            pltpu.VMEM((N_TOK, LANES), jnp.float32),
            pltpu.VMEM((1, LANES), jnp.float32),
            pltpu.VMEM((N_BLK, LANES), jnp.float32),
        ],
    )(x, Wg, bg2d)


def _group_gemm_kernel(hist_ref, xs_ref, w1_ref, b1_ref, wout_ref,
                       bout_ref, out_ref, acc_scr):
    tile_base = pl.program_id(0) * BLK
    starts = [jnp.int32(0)]
    for e in range(N_EXP):
        starts.append(starts[-1] + hist_ref[e])
    r = lax.broadcasted_iota(jnp.int32, (BLK, 1), 0) + tile_base
    for e in range(N_EXP):
        present = jnp.logical_and(starts[e + 1] > tile_base,
                                  starts[e] < tile_base + BLK)

        @pl.when(present)
        def _run(e=e):
            h = jnp.maximum(
                jnp.dot(xs_ref[...], w1_ref[e],
                        preferred_element_type=jnp.float32)
                + b1_ref[e][None, :], 0.0)
            mask = jnp.logical_and(r >= starts[e], r < starts[e + 1])
            acc_scr[...] = jnp.where(mask, h, acc_scr[...])

    out_ref[...] = jnp.dot(acc_scr[...], wout_ref[...],
                           preferred_element_type=jnp.float32) + bout_ref[...]


def _group_gemm(hist128, xs, W1, b1, Wout, bout2d):
    return pl.pallas_call(
        _group_gemm_kernel,
        grid=(N_BLK,),
        in_specs=[
            pl.BlockSpec(memory_space=pltpu.SMEM),
            pl.BlockSpec((BLK, D_IN), lambda i: (i, 0)),
            pl.BlockSpec((N_EXP, D_IN, D_HID), lambda i: (0, 0, 0)),
            pl.BlockSpec((N_EXP, D_HID), lambda i: (0, 0)),
            pl.BlockSpec((D_HID, D_OUT), lambda i: (0, 0)),
            pl.BlockSpec((1, D_OUT), lambda i: (0, 0)),
        ],
        out_specs=pl.BlockSpec((BLK, D_OUT), lambda i: (i, 0)),
        out_shape=jax.ShapeDtypeStruct((N_TOK, D_OUT), jnp.float32),
        scratch_shapes=[pltpu.VMEM((BLK, D_HID), jnp.float32)],
    )(hist128, xs, W1, b1, Wout, bout2d)


_NC = 2
_NS = 16
_NW = _NC * _NS
_RPW = N_TOK // _NW


@functools.lru_cache(maxsize=None)
def _make_sc_permute(d_model, invert):
    mesh = plsc.VectorSubcoreMesh(core_axis_name="c", subcore_axis_name="s")

    @functools.partial(
        pl.kernel, mesh=mesh,
        out_type=jax.ShapeDtypeStruct((N_TOK, d_model), jnp.float32),
        scratch_types=[
            pltpu.VMEM((_RPW,), jnp.int32),
            pltpu.VMEM((_RPW, d_model), jnp.float32),
            pltpu.SemaphoreType.DMA,
        ],
    )
    def k(src_hbm, idx_hbm, out_hbm, idx_v, rows_v, sem):
        wid = lax.axis_index("s") * _NC + lax.axis_index("c")
        base = wid * _RPW
        pltpu.sync_copy(idx_hbm.at[pl.ds(base, _RPW)], idx_v)
        if invert:
            pltpu.async_copy(src_hbm.at[idx_v], rows_v, sem).wait()
            pltpu.sync_copy(rows_v, out_hbm.at[pl.ds(base, _RPW)])
        else:
            pltpu.sync_copy(src_hbm.at[pl.ds(base, _RPW)], rows_v)
            pltpu.async_copy(rows_v, out_hbm.at[idx_v], sem).wait()

    return k


def kernel(x, Wg, bg, W1, b1, Wout, bout):
    rank2d, hist2d = _router_rank(x, Wg, bg.reshape(1, N_EXP))
    rank = rank2d.reshape(N_TOK)
    hist128 = hist2d.reshape(LANES)
    xs = _make_sc_permute(D_IN, False)(x, rank)
    out_sorted = _group_gemm(hist128, xs, W1, b1, Wout,
                             bout.reshape(1, D_OUT))
    return _make_sc_permute(D_OUT, True)(out_sorted, rank)
